# baseline (speedup 1.0000x reference)
import functools

import jax
import jax.numpy as jnp
from jax import lax
from jax.experimental import pallas as pl
from jax.experimental.pallas import tpu as pltpu
from jax.experimental.pallas import tpu_sc as plsc

NUM_Q = 100000
D = 256

CH = 200
NCHUNKS = NUM_Q // CH
NW = 32
PER_W = -(-NCHUNKS // NW)
LAST = NCHUNKS - 1


def _sc_body(src, dst, buf, lsem, ssem):
    wid = lax.axis_index("s") * 2 + lax.axis_index("c")

    def chunk_ds(k):
        j = jnp.minimum(wid + k * NW, LAST)
        return pl.ds(j * CH, CH)

    def load(k, slot):
        c = pltpu.make_async_copy(src.at[chunk_ds(k)], buf.at[slot],
                                  lsem.at[slot])
        c.start()
        return c

    def store(k, slot):
        c = pltpu.make_async_copy(buf.at[slot], dst.at[chunk_ds(k)],
                                  ssem.at[slot])
        c.start()
        return c

    loads = [None] * PER_W
    stores = [None] * PER_W
    loads[0] = load(0, 0)
    for k in range(PER_W):
        slot = k % 2
        if k + 1 < PER_W:
            if k - 1 >= 0:
                stores[k - 1].wait()
            loads[k + 1] = load(k + 1, (k + 1) % 2)
        loads[k].wait()
        stores[k] = store(k, slot)
    stores[PER_W - 2].wait()
    stores[PER_W - 1].wait()


def _sc_copy(table):
    out = jax.ShapeDtypeStruct((NUM_Q, D), jnp.float32)
    mesh = plsc.VectorSubcoreMesh(core_axis_name="c", subcore_axis_name="s")
    k = functools.partial(
        pl.kernel,
        out_type=out,
        mesh=mesh,
        scratch_types=[
            pltpu.VMEM((2, CH, D), jnp.float32),
            pltpu.SemaphoreType.DMA((2,)),
            pltpu.SemaphoreType.DMA((2,)),
        ],
    )(_sc_body)
    return k(table)


BLOCK = 5000
GRID = NUM_Q // BLOCK


def _tc_body(src, dst):
    dst[...] = src[...]


def _tc_copy(table):
    out = jax.ShapeDtypeStruct((NUM_Q, D), jnp.float32)
    spec = pl.BlockSpec((BLOCK, D), lambda i: (i, 0))
    return pl.pallas_call(
        _tc_body,
        grid=(GRID,),
        in_specs=[spec],
        out_specs=spec,
        out_shape=out,
    )(table)


def kernel(batch_size, query_embed_weight, query_pos_weight):
    query_pos = _sc_copy(query_pos_weight)
    query_embed = _sc_copy(query_embed_weight)
    return (query_embed, query_pos)


# speedup vs baseline: 1.1056x; 1.1056x over previous
import functools

import jax
import jax.numpy as jnp
from jax import lax
from jax.experimental import pallas as pl
from jax.experimental.pallas import tpu as pltpu
from jax.experimental.pallas import tpu_sc as plsc

NUM_Q = 100000
D = 256

CH = 160
SLOTS = 3
NCHUNKS = NUM_Q // CH
NW = 32
PER_W = -(-NCHUNKS // NW)
LAST = NCHUNKS - 1


def _sc_body(src, dst, buf, lsem, ssem):
    wid = lax.axis_index("s") * 2 + lax.axis_index("c")

    def chunk_ds(k):
        j = jnp.minimum(wid + k * NW, LAST)
        return pl.ds(j * CH, CH)

    def load(k, slot):
        c = pltpu.make_async_copy(src.at[chunk_ds(k)], buf.at[slot],
                                  lsem.at[slot])
        c.start()
        return c

    def store(k, slot):
        c = pltpu.make_async_copy(buf.at[slot], dst.at[chunk_ds(k)],
                                  ssem.at[slot])
        c.start()
        return c

    loads = [None] * PER_W
    stores = [None] * PER_W
    loads[0] = load(0, 0)
    for k in range(PER_W):
        slot = k % SLOTS
        if k + 1 < PER_W:
            if k + 1 - SLOTS >= 0:
                stores[k + 1 - SLOTS].wait()
            loads[k + 1] = load(k + 1, (k + 1) % SLOTS)
        loads[k].wait()
        stores[k] = store(k, slot)
    for j in range(max(0, PER_W - SLOTS), PER_W):
        stores[j].wait()


def _sc_copy(table):
    out = jax.ShapeDtypeStruct((NUM_Q, D), jnp.float32)
    mesh = plsc.VectorSubcoreMesh(core_axis_name="c", subcore_axis_name="s")
    k = functools.partial(
        pl.kernel,
        out_type=out,
        mesh=mesh,
        scratch_types=[
            pltpu.VMEM((SLOTS, CH, D), jnp.float32),
            pltpu.SemaphoreType.DMA((SLOTS,)),
            pltpu.SemaphoreType.DMA((SLOTS,)),
        ],
    )(_sc_body)
    return k(table)


BLOCK = 5000
GRID = NUM_Q // BLOCK


def _tc_body(src, dst):
    dst[...] = src[...]


def _tc_copy(table):
    out = jax.ShapeDtypeStruct((NUM_Q, D), jnp.float32)
    spec = pl.BlockSpec((BLOCK, D), lambda i: (i, 0))
    return pl.pallas_call(
        _tc_body,
        grid=(GRID,),
        in_specs=[spec],
        out_specs=spec,
        out_shape=out,
    )(table)


def kernel(batch_size, query_embed_weight, query_pos_weight):
    query_pos = _sc_copy(query_pos_weight)
    query_embed = _tc_copy(query_embed_weight)
    return (query_embed, query_pos)


# speedup vs baseline: 1.1087x; 1.0028x over previous
import functools

import jax
import jax.numpy as jnp
from jax import lax
from jax.experimental import pallas as pl
from jax.experimental.pallas import tpu as pltpu
from jax.experimental.pallas import tpu_sc as plsc

NUM_Q = 100000
D = 256

CH = 160
SLOTS = 3
NCHUNKS = NUM_Q // CH
NW = 32
PER_W = -(-NCHUNKS // NW)
LAST = NCHUNKS - 1


def _sc_body(src, dst, buf, lsem, ssem):
    wid = lax.axis_index("s") * 2 + lax.axis_index("c")

    def chunk_ds(k):
        j = jnp.minimum(wid + k * NW, LAST)
        return pl.ds(j * CH, CH)

    def load(k, slot):
        c = pltpu.make_async_copy(src.at[chunk_ds(k)], buf.at[slot],
                                  lsem.at[slot])
        c.start()
        return c

    def store(k, slot):
        c = pltpu.make_async_copy(buf.at[slot], dst.at[chunk_ds(k)],
                                  ssem.at[slot])
        c.start()
        return c

    loads = [None] * PER_W
    stores = [None] * PER_W
    loads[0] = load(0, 0)
    for k in range(PER_W):
        slot = k % SLOTS
        if k + 1 < PER_W:
            if k + 1 - SLOTS >= 0:
                stores[k + 1 - SLOTS].wait()
            loads[k + 1] = load(k + 1, (k + 1) % SLOTS)
        loads[k].wait()
        stores[k] = store(k, slot)
    for j in range(max(0, PER_W - SLOTS), PER_W):
        stores[j].wait()


def _sc_copy(table):
    out = jax.ShapeDtypeStruct((NUM_Q, D), jnp.float32)
    mesh = plsc.VectorSubcoreMesh(core_axis_name="c", subcore_axis_name="s")
    k = functools.partial(
        pl.kernel,
        out_type=out,
        mesh=mesh,
        scratch_types=[
            pltpu.VMEM((SLOTS, CH, D), jnp.float32),
            pltpu.SemaphoreType.DMA((SLOTS,)),
            pltpu.SemaphoreType.DMA((SLOTS,)),
        ],
    )(_sc_body)
    return k(table)


BLOCK = 5000
GRID = NUM_Q // BLOCK


def _tc_body(src, dst):
    dst[...] = src[...]


def _tc_copy(table):
    out = jax.ShapeDtypeStruct((NUM_Q, D), jnp.float32)
    spec = pl.BlockSpec((BLOCK, D), lambda i: (i, 0))
    return pl.pallas_call(
        _tc_body,
        grid=(GRID,),
        in_specs=[spec],
        out_specs=spec,
        out_shape=out,
    )(table)


def kernel(batch_size, query_embed_weight, query_pos_weight):
    query_embed = _tc_copy(query_embed_weight)
    query_pos = _sc_copy(query_pos_weight)
    return (query_embed, query_pos)


# speedup vs baseline: 1.1093x; 1.0005x over previous
import functools

import jax
import jax.numpy as jnp
from jax import lax
from jax.experimental import pallas as pl
from jax.experimental.pallas import tpu as pltpu
from jax.experimental.pallas import tpu_sc as plsc

NUM_Q = 100000
D = 256

CH = 200
SLOTS = 2
NCHUNKS = NUM_Q // CH
NW = 32
PER_W = -(-NCHUNKS // NW)
LAST = NCHUNKS - 1


def _sc_body(src, dst, buf, lsem, ssem):
    wid = lax.axis_index("s") * 2 + lax.axis_index("c")

    def chunk_ds(k):
        j = jnp.minimum(wid + k * NW, LAST)
        return pl.ds(j * CH, CH)

    def load(k, slot):
        c = pltpu.make_async_copy(src.at[chunk_ds(k)], buf.at[slot],
                                  lsem.at[slot])
        c.start()
        return c

    def store(k, slot):
        c = pltpu.make_async_copy(buf.at[slot], dst.at[chunk_ds(k)],
                                  ssem.at[slot])
        c.start()
        return c

    loads = [None] * PER_W
    stores = [None] * PER_W
    loads[0] = load(0, 0)
    for k in range(PER_W):
        slot = k % SLOTS
        if k + 1 < PER_W:
            if k + 1 - SLOTS >= 0:
                stores[k + 1 - SLOTS].wait()
            loads[k + 1] = load(k + 1, (k + 1) % SLOTS)
        loads[k].wait()
        stores[k] = store(k, slot)
    for j in range(max(0, PER_W - SLOTS), PER_W):
        stores[j].wait()


def _sc_copy(table):
    out = jax.ShapeDtypeStruct((NUM_Q, D), jnp.float32)
    mesh = plsc.VectorSubcoreMesh(core_axis_name="c", subcore_axis_name="s")
    k = functools.partial(
        pl.kernel,
        out_type=out,
        mesh=mesh,
        scratch_types=[
            pltpu.VMEM((SLOTS, CH, D), jnp.float32),
            pltpu.SemaphoreType.DMA((SLOTS,)),
            pltpu.SemaphoreType.DMA((SLOTS,)),
        ],
    )(_sc_body)
    return k(table)


BLOCK = 5000
GRID = NUM_Q // BLOCK


def _tc_body(src, dst):
    dst[...] = src[...]


def _tc_copy(table):
    out = jax.ShapeDtypeStruct((NUM_Q, D), jnp.float32)
    spec = pl.BlockSpec((BLOCK, D), lambda i: (i, 0))
    return pl.pallas_call(
        _tc_body,
        grid=(GRID,),
        in_specs=[spec],
        out_specs=spec,
        out_shape=out,
    )(table)


def kernel(batch_size, query_embed_weight, query_pos_weight):
    query_embed = _tc_copy(query_embed_weight)
    query_pos = _sc_copy(query_pos_weight)
    return (query_embed, query_pos)


# speedup vs baseline: 1.1177x; 1.0076x over previous
import functools

import jax
import jax.numpy as jnp
from jax import lax
from jax.experimental import pallas as pl
from jax.experimental.pallas import tpu as pltpu
from jax.experimental.pallas import tpu_sc as plsc

NUM_Q = 100000
D = 256

CH = 200
SLOTS = 2
NCHUNKS = NUM_Q // CH
NW = 32
PER_W = -(-NCHUNKS // NW)
LAST = NCHUNKS - 1


EXTRA = NCHUNKS - (NCHUNKS // NW) * NW


def _sc_body(src, dst, buf, lsem, ssem):
    wid = lax.axis_index("s") * 2 + lax.axis_index("c")
    has_extra = wid < EXTRA

    def chunk_ds(k):
        j = jnp.minimum(wid + k * NW, LAST)
        return pl.ds(j * CH, CH)

    def run(k, op):
        if k == PER_W - 1:
            @pl.when(has_extra)
            def _():
                op()
        else:
            op()

    def load(k, slot):
        c = pltpu.make_async_copy(src.at[chunk_ds(k)], buf.at[slot],
                                  lsem.at[slot])
        run(k, c.start)
        return c

    def store(k, slot):
        c = pltpu.make_async_copy(buf.at[slot], dst.at[chunk_ds(k)],
                                  ssem.at[slot])
        run(k, c.start)
        return c

    loads = [None] * PER_W
    stores = [None] * PER_W
    loads[0] = load(0, 0)
    for k in range(PER_W):
        slot = k % SLOTS
        if k + 1 < PER_W:
            if k + 1 - SLOTS >= 0:
                stores[k + 1 - SLOTS].wait()
            loads[k + 1] = load(k + 1, (k + 1) % SLOTS)
        run(k, loads[k].wait)
        stores[k] = store(k, slot)
    for j in range(max(0, PER_W - SLOTS), PER_W):
        run(j, stores[j].wait)


def _sc_copy(table):
    out = jax.ShapeDtypeStruct((NUM_Q, D), jnp.float32)
    mesh = plsc.VectorSubcoreMesh(core_axis_name="c", subcore_axis_name="s")
    k = functools.partial(
        pl.kernel,
        out_type=out,
        mesh=mesh,
        scratch_types=[
            pltpu.VMEM((SLOTS, CH, D), jnp.float32),
            pltpu.SemaphoreType.DMA((SLOTS,)),
            pltpu.SemaphoreType.DMA((SLOTS,)),
        ],
    )(_sc_body)
    return k(table)


BLOCK = 5000
GRID = NUM_Q // BLOCK


def _tc_body(src, dst):
    dst[...] = src[...]


def _tc_copy(table):
    out = jax.ShapeDtypeStruct((NUM_Q, D), jnp.float32)
    spec = pl.BlockSpec((BLOCK, D), lambda i: (i, 0))
    return pl.pallas_call(
        _tc_body,
        grid=(GRID,),
        in_specs=[spec],
        out_specs=spec,
        out_shape=out,
    )(table)


def kernel(batch_size, query_embed_weight, query_pos_weight):
    query_embed = _tc_copy(query_embed_weight)
    query_pos = _sc_copy(query_pos_weight)
    return (query_embed, query_pos)
